# baseline (speedup 1.0000x reference)
import functools

import jax
import jax.numpy as jnp
from jax import lax
from jax.experimental import pallas as pl
from jax.experimental.pallas import tpu as pltpu
from jax.experimental.pallas import tpu_sc as plsc

NUM_EMB = 1000
D = 64
BATCH = 16384
HIST = 200
N = BATCH * HIST

NC = 2
NS = 16
NW = NC * NS
PER_W = N // NW
SUB = 128
CHUNK = 256
K = CHUNK // SUB
NCH = PER_W // CHUNK
NBUF = 4
GRP = NCH // NBUF


def _table_body(interp_ref, emb_ref, out_ref):
    out_ref[...] = jnp.dot(interp_ref[...], emb_ref[...],
                           preferred_element_type=jnp.float32)


def _build_table(interp, emb):
    return pl.pallas_call(
        _table_body,
        out_shape=jax.ShapeDtypeStruct((NUM_EMB, D), jnp.float32),
    )(interp, emb)


_mesh = plsc.VectorSubcoreMesh(core_axis_name="c", subcore_axis_name="s")


@functools.partial(
    pl.kernel,
    mesh=_mesh,
    compiler_params=pltpu.CompilerParams(use_tc_tiling_on_sc=False),
    out_type=jax.ShapeDtypeStruct((N, D), jnp.float32),
    scratch_types=(
        [pltpu.VMEM((NBUF, K, SUB), jnp.int32),
         pltpu.VMEM((NBUF * CHUNK, D), jnp.float32)]
        + [pltpu.SemaphoreType.DMA] * (2 * NBUF)
    ),
)
def _sc_gather(table_hbm, idx_hbm, out_hbm, idx_v, rows_v, *sems):
    gsem = sems[:NBUF]
    osem = sems[NBUF:]
    wid = lax.axis_index("s") * NC + lax.axis_index("c")
    idx_row0 = pl.multiple_of(wid * (PER_W // SUB), 8)
    row0 = pl.multiple_of(wid * PER_W, 8)

    def fire_gather(g, b):
        pltpu.sync_copy(idx_hbm.at[pl.ds(idx_row0 + g * K, K)], idx_v.at[b])
        for k in range(K):
            pltpu.async_copy(table_hbm.at[idx_v.at[b, k]],
                             rows_v.at[pl.ds(b * CHUNK + k * SUB, SUB)],
                             gsem[b])

    def drain_gather(g, b):
        for k in range(K):
            pltpu.make_async_copy(table_hbm.at[idx_v.at[b, k]],
                                  rows_v.at[pl.ds(b * CHUNK + k * SUB, SUB)],
                                  gsem[b]).wait()

    def fire_out(g, b):
        pltpu.async_copy(rows_v.at[pl.ds(b * CHUNK, CHUNK)],
                         out_hbm.at[pl.ds(row0 + g * CHUNK, CHUNK)],
                         osem[b])

    def wait_out(g, b):
        pltpu.make_async_copy(rows_v.at[pl.ds(b * CHUNK, CHUNK)],
                              out_hbm.at[pl.ds(row0 + g * CHUNK, CHUNK)],
                              osem[b]).wait()

    fire_gather(0, 0)
    fire_gather(1, 1)

    for b in range(NBUF):
        g = b
        drain_gather(g, b)
        fire_out(g, b)
        if b >= 2:
            wait_out(g - 2, (b + 2) % NBUF)
        fire_gather(g + 2, (b + 2) % NBUF)

    def body(gg, carry):
        for b in range(NBUF):
            g = gg * NBUF + b
            drain_gather(g, b)
            fire_out(g, b)
            wait_out(g - 2, (b + 2) % NBUF)
            fire_gather(g + 2, (b + 2) % NBUF)
        return carry

    lax.fori_loop(1, GRP - 1, body, 0)

    for b in range(NBUF):
        g = NCH - NBUF + b
        drain_gather(g, b)
        fire_out(g, b)
        wait_out(g - 2, (b + 2) % NBUF)
        if b < 2:
            fire_gather(g + 2, (b + 2) % NBUF)

    wait_out(NCH - 2, (NCH - 2) % NBUF)
    wait_out(NCH - 1, (NCH - 1) % NBUF)


def kernel(index_tensor, embedding_matrix, interpolation_matrix):
    table = _build_table(interpolation_matrix, embedding_matrix)
    idx = index_tensor.reshape(N // SUB, SUB).astype(jnp.int32)
    out = _sc_gather(table, idx)
    return out.reshape(BATCH, HIST, D)
